# baseline (speedup 1.0000x reference)
import functools

import jax
import jax.numpy as jnp
from jax.experimental import pallas as pl
from jax.experimental.pallas import tpu as pltpu

N = 10000
H = 512
N_REPEAT = 7
EPS = 1e-5
BM = 1000
T = N // BM


def _chunked_bf16(xn):
    xc = xn.reshape(BM, H // 128, 128).transpose(1, 0, 2)
    return xc


def _prologue_body(xin_ref, w1_ref, b1_ref, o_ref, oc_ref):
    h = jnp.dot(xin_ref[:], w1_ref[:], preferred_element_type=jnp.float32)
    xn = jnp.maximum(h + b1_ref[:], 0.0)
    o_ref[:] = xn
    oc_ref[:] = _chunked_bf16(xn)


def _round_a_body(x_ref, nei_ref, w2a_ref, w2b_ref, b2_ref, h_ref, s_ref):
    h = jnp.dot(x_ref[:], w2a_ref[:], preferred_element_type=jnp.float32)
    h = h + jnp.dot(nei_ref[:], w2b_ref[:], preferred_element_type=jnp.float32)
    h = h + b2_ref[:]
    h_ref[:] = h

    @pl.when(pl.program_id(0) == 0)
    def _():
        s_ref[:] = jnp.zeros_like(s_ref)

    s_ref[:] += jnp.sum(h, axis=0, keepdims=True)


def _round_v_body(h_ref, s_ref, v_ref):
    @pl.when(pl.program_id(0) == 0)
    def _():
        v_ref[:] = jnp.zeros_like(v_ref)

    c = h_ref[:] - s_ref[:] / N
    v_ref[:] += jnp.sum(c * c, axis=0, keepdims=True)


def _round_b_body(h_ref, s_ref, v_ref, g_ref, bt_ref, o_ref, oc_ref):
    mean = s_ref[:] / N
    var = v_ref[:] / N
    xn = jnp.maximum(
        g_ref[:] * (h_ref[:] - mean) / jnp.sqrt(var + EPS) + bt_ref[:], 0.0)
    o_ref[:] = xn
    oc_ref[:] = _chunked_bf16(xn)


def _epilogue_body(x_ref, x0_ref, xin_ref, w3_ref, w4_ref, o_ref):
    xf = jnp.concatenate([x_ref[:], x0_ref[:]], axis=1)
    ratio = jnp.dot(xf, w3_ref[:], preferred_element_type=jnp.float32)
    delta = jnp.dot(xf, w4_ref[:], preferred_element_type=jnp.float32)
    o_ref[:] = ratio * xin_ref[:, 0:1] + delta


def _dense_prologue(xin_pad, w1t, b1):
    return pl.pallas_call(
        _prologue_body,
        grid=(T,),
        in_specs=[
            pl.BlockSpec((BM, 16), lambda t: (t, 0)),
            pl.BlockSpec((16, H), lambda t: (0, 0)),
            pl.BlockSpec((1, H), lambda t: (0, 0)),
        ],
        out_specs=[
            pl.BlockSpec((BM, H), lambda t: (t, 0)),
            pl.BlockSpec((H // 128, BM, 128), lambda t: (0, t, 0)),
        ],
        out_shape=[
            jax.ShapeDtypeStruct((N, H), jnp.float32),
            jax.ShapeDtypeStruct((H // 128, N, 128), jnp.float32),
        ],
    )(xin_pad, w1t, b1)


def _dense_round(x, nei, w2ta, w2tb, b2, gamma, beta):
    h, s = pl.pallas_call(
        _round_a_body,
        grid=(T,),
        in_specs=[
            pl.BlockSpec((BM, H), lambda t: (t, 0)),
            pl.BlockSpec((BM, H), lambda t: (t, 0)),
            pl.BlockSpec((H, H), lambda t: (0, 0)),
            pl.BlockSpec((H, H), lambda t: (0, 0)),
            pl.BlockSpec((1, H), lambda t: (0, 0)),
        ],
        out_specs=[
            pl.BlockSpec((BM, H), lambda t: (t, 0)),
            pl.BlockSpec((1, H), lambda t: (0, 0)),
        ],
        out_shape=[
            jax.ShapeDtypeStruct((N, H), jnp.float32),
            jax.ShapeDtypeStruct((1, H), jnp.float32),
        ],
    )(x, nei, w2ta, w2tb, b2)
    v = pl.pallas_call(
        _round_v_body,
        grid=(T,),
        in_specs=[
            pl.BlockSpec((BM, H), lambda t: (t, 0)),
            pl.BlockSpec((1, H), lambda t: (0, 0)),
        ],
        out_specs=pl.BlockSpec((1, H), lambda t: (0, 0)),
        out_shape=jax.ShapeDtypeStruct((1, H), jnp.float32),
    )(h, s)
    return pl.pallas_call(
        _round_b_body,
        grid=(T,),
        in_specs=[
            pl.BlockSpec((BM, H), lambda t: (t, 0)),
            pl.BlockSpec((1, H), lambda t: (0, 0)),
            pl.BlockSpec((1, H), lambda t: (0, 0)),
            pl.BlockSpec((1, H), lambda t: (0, 0)),
            pl.BlockSpec((1, H), lambda t: (0, 0)),
        ],
        out_specs=[
            pl.BlockSpec((BM, H), lambda t: (t, 0)),
            pl.BlockSpec((H // 128, BM, 128), lambda t: (0, t, 0)),
        ],
        out_shape=[
            jax.ShapeDtypeStruct((N, H), jnp.float32),
            jax.ShapeDtypeStruct((H // 128, N, 128), jnp.float32),
        ],
    )(h, s, v, gamma, beta)


def _dense_epilogue(x, x0, xin_pad, w3p, w4p):
    return pl.pallas_call(
        _epilogue_body,
        grid=(T,),
        in_specs=[
            pl.BlockSpec((BM, H), lambda t: (t, 0)),
            pl.BlockSpec((BM, H), lambda t: (t, 0)),
            pl.BlockSpec((BM, 16), lambda t: (t, 0)),
            pl.BlockSpec((2 * H, 8), lambda t: (0, 0)),
            pl.BlockSpec((2 * H, 8), lambda t: (0, 0)),
        ],
        out_specs=pl.BlockSpec((BM, 8), lambda t: (t, 0)),
        out_shape=jax.ShapeDtypeStruct((N, 8), jnp.float32),
    )(x, x0, xin_pad, w3p, w4p)


NC = 2
NS = 16
L = 16
FC = 128
NCH = H // FC
CPS = NCH // NC
EB = 128
E_PAD = 163840
NB = E_PAD // (NS * EB)
NBH = NB // 2
EPT = E_PAD // NS
N_ACC = 10240
ROWS_PT = N_ACC // NS

_sc_mesh = None


def _get_mesh():
    global _sc_mesh
    if _sc_mesh is None:
        from jax.experimental.pallas import tpu_sc as plsc
        _sc_mesh = plsc.VectorSubcoreMesh(core_axis_name="c", subcore_axis_name="s")
    return _sc_mesh


def _nei_body(xf_ref, pk_ref, wg_ref, out_ref,
              src_v, dst_v, w_v, rows_v, rows2_v, accum, sem0, sem1, ssem0, ssem1):
    from jax import lax
    from jax.experimental.pallas import tpu_sc as plsc
    cid = lax.axis_index("c")
    sid = lax.axis_index("s")
    base = sid * ROWS_PT

    def _zero_rows(r, carry):
        for q in range(FC // L):
            rows_v[r, pl.ds(q * L, L)] = jnp.zeros((L,), jnp.float32)
        return carry

    def _zero_accum_slice():
        lax.fori_loop(0, EB, _zero_rows, 0)
        for k in range(ROWS_PT // EB):
            pltpu.sync_copy(rows_v,
                            accum.at[pl.ds(base + k * EB, EB)])

    _zero_accum_slice()
    plsc.subcore_barrier()

    def _weight(b, buf):
        def _wgroup(g, carry2):
            wv = w_v[b, pl.ds(g * L, L)]
            for es in range(L):
                wb = jnp.broadcast_to(wv[es], (L,))
                r = g * L + es
                for q in range(FC // L):
                    buf[r, pl.ds(q * L, L)] = buf[r, pl.ds(q * L, L)] * wb
            return carry2

        lax.fori_loop(0, EB // L, _wgroup, 0)

    for j in range(CPS):
        c = cid * CPS + j
        off = (cid * CPS + j) * jnp.int32(N)
        for h in range(NB // NBH):
            pltpu.sync_copy(pk_ref.at[sid, pl.ds(h * NBH, NBH)], dst_v)
            pltpu.sync_copy(wg_ref.at[sid, pl.ds(h * NBH, NBH)], w_v)

            def _unpack_idx(b, carry):
                for g in range(EB // L):
                    v = dst_v[b, pl.ds(g * L, L)]
                    src_v[b, pl.ds(g * L, L)] = lax.shift_right_logical(v, 16) + off
                    dst_v[b, pl.ds(g * L, L)] = v & jnp.int32(0xFFFF)
                return carry

            lax.fori_loop(0, NBH, _unpack_idx, 0)
            pltpu.async_copy(xf_ref.at[src_v.at[0]], rows_v, sem0)
            pltpu.async_copy(xf_ref.at[src_v.at[1]], rows2_v, sem1)

            def _pair(i, carry):
                b0 = 2 * i
                pltpu.make_async_copy(xf_ref.at[src_v.at[b0]], rows_v, sem0).wait()
                _weight(b0, rows_v)
                pltpu.async_copy(rows_v, accum.at[dst_v.at[b0]], ssem0, add=True)
                pltpu.make_async_copy(xf_ref.at[src_v.at[b0 + 1]], rows2_v, sem1).wait()
                _weight(b0 + 1, rows2_v)
                pltpu.async_copy(rows2_v, accum.at[dst_v.at[b0 + 1]], ssem1, add=True)

                @pl.when(i < NBH // 2 - 1)
                def _():
                    pltpu.make_async_copy(rows_v, accum.at[dst_v.at[b0]], ssem0).wait()
                    pltpu.async_copy(xf_ref.at[src_v.at[b0 + 2]], rows_v, sem0)
                    pltpu.make_async_copy(rows2_v, accum.at[dst_v.at[b0 + 1]], ssem1).wait()
                    pltpu.async_copy(xf_ref.at[src_v.at[b0 + 3]], rows2_v, sem1)
                return carry

            lax.fori_loop(0, NBH // 2, _pair, 0)
            pltpu.make_async_copy(rows_v, accum.at[dst_v.at[NBH - 2]], ssem0).wait()
            pltpu.make_async_copy(rows2_v, accum.at[dst_v.at[NBH - 1]], ssem1).wait()
        plsc.subcore_barrier()

        @pl.when(sid < NS - 1)
        def _():
            pltpu.sync_copy(accum.at[pl.ds(base, ROWS_PT)],
                            out_ref.at[c, pl.ds(base, ROWS_PT)])

        @pl.when(sid == NS - 1)
        def _():
            pltpu.sync_copy(accum.at[pl.ds(base, N - (NS - 1) * ROWS_PT)],
                            out_ref.at[c, pl.ds(base, N - (NS - 1) * ROWS_PT)])

        if j < CPS - 1:
            _zero_accum_slice()
        plsc.subcore_barrier()


def _nei_sum_sc(xf, pk, wg, interpret=False):
    kern = functools.partial(
        pl.kernel,
        out_type=jax.ShapeDtypeStruct((NCH, N, FC), jnp.float32),
        mesh=_get_mesh(),
        scratch_types=[
            pltpu.VMEM((NBH, EB), jnp.int32),
            pltpu.VMEM((NBH, EB), jnp.int32),
            pltpu.VMEM((NBH, EB), jnp.float32),
            pltpu.VMEM((EB, FC), jnp.float32),
            pltpu.VMEM((EB, FC), jnp.float32),
            pltpu.VMEM_SHARED((N_ACC, FC), jnp.float32),
            pltpu.SemaphoreType.DMA,
            pltpu.SemaphoreType.DMA,
            pltpu.SemaphoreType.DMA,
            pltpu.SemaphoreType.DMA,
        ],
        interpret=interpret,
    )(_nei_body)
    return kern(xf, pk, wg)


def _edge_plumbing(src, dst, edge_weight):
    pad = E_PAD - src.shape[0]
    src_p = jnp.pad(src, (0, pad)).reshape(NS, NB, EB)
    dst_p = jnp.pad(dst, (0, pad)).reshape(NS, NB, EB)
    pk = jnp.left_shift(src_p, 16) | dst_p
    wg = jnp.pad(edge_weight, (0, pad)).reshape(NS, NB, EB)
    return pk, wg


def _nei_sum(x, src, dst, edge_weight):
    gathered = jnp.take(x, src, axis=0) * edge_weight[:, None]
    return jax.ops.segment_sum(gathered, dst, num_segments=N)


def kernel(x_in, edge_index, edge_weight, W1, b1, W2, b2, gamma, beta, W3, b3, W4, b4):
    src = edge_index[0]
    dst = edge_index[1]
    xin_pad = jnp.pad(x_in, ((0, 0), (0, 1)))
    w1t = jnp.pad(W1.T, ((0, 1), (0, 0)))
    w2t = W2.T
    w2ta = w2t[:H]
    w2tb = w2t[H:]
    b1r = b1.reshape(1, H)
    b2r = b2.reshape(1, H)
    gr = gamma.reshape(1, H)
    br = beta.reshape(1, H)
    w3p = jnp.pad(W3.T, ((0, 0), (0, 7)))
    w4p = jnp.pad(W4.T, ((0, 0), (0, 7)))

    pk, wg = _edge_plumbing(src, dst, edge_weight)

    x, xc = _dense_prologue(xin_pad, w1t, b1r)
    x_init = x
    for _ in range(N_REPEAT):
        xf = xc.reshape(NCH * N, FC)
        neic = _nei_sum_sc(xf, pk, wg)
        nei = neic.transpose(1, 0, 2).reshape(N, H)
        x, xc = _dense_round(x, nei, w2ta, w2tb, b2r, gr, br)
    y = _dense_epilogue(x, x_init, xin_pad, w3p, w4p)
    out = y[:, 0:1] + b3.reshape(1, 1) * x_in[:, 0:1] + b4.reshape(1, 1)
    return out


# speedup vs baseline: 1.1038x; 1.1038x over previous
import functools

import jax
import jax.numpy as jnp
from jax.experimental import pallas as pl
from jax.experimental.pallas import tpu as pltpu

N = 10000
H = 512
N_REPEAT = 7
EPS = 1e-5
BM = 1000
T = N // BM


def _chunked_bf16(xn):
    xc = xn.reshape(BM, H // 128, 128).transpose(1, 0, 2)
    return xc


def _prologue_body(xin_ref, w1_ref, b1_ref, o_ref, oc_ref):
    h = jnp.dot(xin_ref[:], w1_ref[:], preferred_element_type=jnp.float32)
    xn = jnp.maximum(h + b1_ref[:], 0.0)
    o_ref[:] = xn
    oc_ref[:] = _chunked_bf16(xn)


def _round_a_body(x_ref, nei_ref, w2a_ref, w2b_ref, b2_ref, h_ref, s_ref):
    h = jnp.dot(x_ref[:], w2a_ref[:], preferred_element_type=jnp.float32)
    h = h + jnp.dot(nei_ref[:], w2b_ref[:], preferred_element_type=jnp.float32)
    h = h + b2_ref[:]
    h_ref[:] = h

    @pl.when(pl.program_id(0) == 0)
    def _():
        s_ref[:] = jnp.zeros_like(s_ref)

    s_ref[:] += jnp.sum(h, axis=0, keepdims=True)


def _round_v_body(h_ref, s_ref, v_ref):
    @pl.when(pl.program_id(0) == 0)
    def _():
        v_ref[:] = jnp.zeros_like(v_ref)

    c = h_ref[:] - s_ref[:] / N
    v_ref[:] += jnp.sum(c * c, axis=0, keepdims=True)


def _round_b_body(h_ref, s_ref, v_ref, g_ref, bt_ref, o_ref, oc_ref):
    mean = s_ref[:] / N
    var = v_ref[:] / N
    xn = jnp.maximum(
        g_ref[:] * (h_ref[:] - mean) / jnp.sqrt(var + EPS) + bt_ref[:], 0.0)
    o_ref[:] = xn
    oc_ref[:] = _chunked_bf16(xn)


def _epilogue_body(x_ref, x0_ref, xin_ref, w3_ref, w4_ref, o_ref):
    xf = jnp.concatenate([x_ref[:], x0_ref[:]], axis=1)
    ratio = jnp.dot(xf, w3_ref[:], preferred_element_type=jnp.float32)
    delta = jnp.dot(xf, w4_ref[:], preferred_element_type=jnp.float32)
    o_ref[:] = ratio * xin_ref[:, 0:1] + delta


def _dense_prologue(xin_pad, w1t, b1):
    return pl.pallas_call(
        _prologue_body,
        grid=(T,),
        in_specs=[
            pl.BlockSpec((BM, 16), lambda t: (t, 0)),
            pl.BlockSpec((16, H), lambda t: (0, 0)),
            pl.BlockSpec((1, H), lambda t: (0, 0)),
        ],
        out_specs=[
            pl.BlockSpec((BM, H), lambda t: (t, 0)),
            pl.BlockSpec((H // 128, BM, 128), lambda t: (0, t, 0)),
        ],
        out_shape=[
            jax.ShapeDtypeStruct((N, H), jnp.float32),
            jax.ShapeDtypeStruct((H // 128, N, 128), jnp.float32),
        ],
    )(xin_pad, w1t, b1)


def _dense_round(x, nei, w2ta, w2tb, b2, gamma, beta):
    h, s = pl.pallas_call(
        _round_a_body,
        grid=(T,),
        in_specs=[
            pl.BlockSpec((BM, H), lambda t: (t, 0)),
            pl.BlockSpec((BM, H), lambda t: (t, 0)),
            pl.BlockSpec((H, H), lambda t: (0, 0)),
            pl.BlockSpec((H, H), lambda t: (0, 0)),
            pl.BlockSpec((1, H), lambda t: (0, 0)),
        ],
        out_specs=[
            pl.BlockSpec((BM, H), lambda t: (t, 0)),
            pl.BlockSpec((1, H), lambda t: (0, 0)),
        ],
        out_shape=[
            jax.ShapeDtypeStruct((N, H), jnp.float32),
            jax.ShapeDtypeStruct((1, H), jnp.float32),
        ],
    )(x, nei, w2ta, w2tb, b2)
    v = pl.pallas_call(
        _round_v_body,
        grid=(T,),
        in_specs=[
            pl.BlockSpec((BM, H), lambda t: (t, 0)),
            pl.BlockSpec((1, H), lambda t: (0, 0)),
        ],
        out_specs=pl.BlockSpec((1, H), lambda t: (0, 0)),
        out_shape=jax.ShapeDtypeStruct((1, H), jnp.float32),
    )(h, s)
    return pl.pallas_call(
        _round_b_body,
        grid=(T,),
        in_specs=[
            pl.BlockSpec((BM, H), lambda t: (t, 0)),
            pl.BlockSpec((1, H), lambda t: (0, 0)),
            pl.BlockSpec((1, H), lambda t: (0, 0)),
            pl.BlockSpec((1, H), lambda t: (0, 0)),
            pl.BlockSpec((1, H), lambda t: (0, 0)),
        ],
        out_specs=[
            pl.BlockSpec((BM, H), lambda t: (t, 0)),
            pl.BlockSpec((H // 128, BM, 128), lambda t: (0, t, 0)),
        ],
        out_shape=[
            jax.ShapeDtypeStruct((N, H), jnp.float32),
            jax.ShapeDtypeStruct((H // 128, N, 128), jnp.float32),
        ],
    )(h, s, v, gamma, beta)


def _dense_epilogue(x, x0, xin_pad, w3p, w4p):
    return pl.pallas_call(
        _epilogue_body,
        grid=(T,),
        in_specs=[
            pl.BlockSpec((BM, H), lambda t: (t, 0)),
            pl.BlockSpec((BM, H), lambda t: (t, 0)),
            pl.BlockSpec((BM, 16), lambda t: (t, 0)),
            pl.BlockSpec((2 * H, 8), lambda t: (0, 0)),
            pl.BlockSpec((2 * H, 8), lambda t: (0, 0)),
        ],
        out_specs=pl.BlockSpec((BM, 8), lambda t: (t, 0)),
        out_shape=jax.ShapeDtypeStruct((N, 8), jnp.float32),
    )(x, x0, xin_pad, w3p, w4p)


NC = 2
NS = 16
L = 16
FC = 128
NCH = H // FC
CPS = NCH // NC
EB = 128
E_PAD = 163840
NB = E_PAD // (NS * EB)
NBH = NB // 2
EPT = E_PAD // NS
N_ACC = 10240
ROWS_PT = N_ACC // NS

_sc_mesh = None


def _get_mesh():
    global _sc_mesh
    if _sc_mesh is None:
        from jax.experimental.pallas import tpu_sc as plsc
        _sc_mesh = plsc.VectorSubcoreMesh(core_axis_name="c", subcore_axis_name="s")
    return _sc_mesh


def _nei_body(xf_ref, pk_ref, wg_ref, out_ref,
              src_v, dst_v, w_v, rows_v, rows2_v, accum, sem0, sem1):
    from jax import lax
    from jax.experimental.pallas import tpu_sc as plsc
    cid = lax.axis_index("c")
    sid = lax.axis_index("s")
    base = sid * ROWS_PT

    def _zero_rows(r, carry):
        for q in range(FC // L):
            rows_v[r, pl.ds(q * L, L)] = jnp.zeros((L,), jnp.float32)
        return carry

    def _zero_accum_slice():
        lax.fori_loop(0, EB, _zero_rows, 0)
        for k in range(ROWS_PT // EB):
            pltpu.sync_copy(rows_v,
                            accum.at[pl.ds(base + k * EB, EB)])

    _zero_accum_slice()
    plsc.subcore_barrier()

    def _weight(b, buf):
        def _wgroup(g, carry2):
            wv = w_v[b, pl.ds(g * L, L)]
            for es in range(L):
                wb = jnp.broadcast_to(wv[es], (L,))
                r = g * L + es
                for q in range(FC // L):
                    buf[r, pl.ds(q * L, L)] = buf[r, pl.ds(q * L, L)] * wb
            return carry2

        lax.fori_loop(0, EB // L, _wgroup, 0)

    for j in range(CPS):
        c = cid * CPS + j
        off = (cid * CPS + j) * jnp.int32(N)
        for h in range(NB // NBH):
            pltpu.sync_copy(pk_ref.at[sid, pl.ds(h * NBH, NBH)], dst_v)
            pltpu.sync_copy(wg_ref.at[sid, pl.ds(h * NBH, NBH)], w_v)

            def _unpack_idx(b, carry):
                for g in range(EB // L):
                    v = dst_v[b, pl.ds(g * L, L)]
                    src_v[b, pl.ds(g * L, L)] = lax.shift_right_logical(v, 16) + off
                    dst_v[b, pl.ds(g * L, L)] = v & jnp.int32(0xFFFF)
                return carry

            lax.fori_loop(0, NBH, _unpack_idx, 0)
            pltpu.async_copy(xf_ref.at[src_v.at[0]], rows_v, sem0)

            def _pair(i, carry):
                b0 = 2 * i
                g1 = pltpu.async_copy(xf_ref.at[src_v.at[b0 + 1]], rows2_v, sem1)
                pltpu.make_async_copy(xf_ref.at[src_v.at[b0]], rows_v, sem0).wait()
                _weight(b0, rows_v)
                pltpu.sync_copy(rows_v, accum.at[dst_v.at[b0]], add=True)

                @pl.when(i < NBH // 2 - 1)
                def _():
                    pltpu.async_copy(xf_ref.at[src_v.at[b0 + 2]], rows_v, sem0)

                g1.wait()
                _weight(b0 + 1, rows2_v)
                pltpu.sync_copy(rows2_v, accum.at[dst_v.at[b0 + 1]], add=True)
                return carry

            lax.fori_loop(0, NBH // 2, _pair, 0)
        plsc.subcore_barrier()

        @pl.when(sid < NS - 1)
        def _():
            pltpu.sync_copy(accum.at[pl.ds(base, ROWS_PT)],
                            out_ref.at[c, pl.ds(base, ROWS_PT)])

        @pl.when(sid == NS - 1)
        def _():
            pltpu.sync_copy(accum.at[pl.ds(base, N - (NS - 1) * ROWS_PT)],
                            out_ref.at[c, pl.ds(base, N - (NS - 1) * ROWS_PT)])

        if j < CPS - 1:
            _zero_accum_slice()
        plsc.subcore_barrier()


def _nei_sum_sc(xf, pk, wg, interpret=False):
    kern = functools.partial(
        pl.kernel,
        out_type=jax.ShapeDtypeStruct((NCH, N, FC), jnp.float32),
        mesh=_get_mesh(),
        scratch_types=[
            pltpu.VMEM((NBH, EB), jnp.int32),
            pltpu.VMEM((NBH, EB), jnp.int32),
            pltpu.VMEM((NBH, EB), jnp.float32),
            pltpu.VMEM((EB, FC), jnp.float32),
            pltpu.VMEM((EB, FC), jnp.float32),
            pltpu.VMEM_SHARED((N_ACC, FC), jnp.float32),
            pltpu.SemaphoreType.DMA,
            pltpu.SemaphoreType.DMA,
        ],
        interpret=interpret,
    )(_nei_body)
    return kern(xf, pk, wg)


def _edge_plumbing(src, dst, edge_weight):
    pad = E_PAD - src.shape[0]
    src_p = jnp.pad(src, (0, pad)).reshape(NS, NB, EB)
    dst_p = jnp.pad(dst, (0, pad)).reshape(NS, NB, EB)
    pk = jnp.left_shift(src_p, 16) | dst_p
    wg = jnp.pad(edge_weight, (0, pad)).reshape(NS, NB, EB)
    return pk, wg


def _nei_sum(x, src, dst, edge_weight):
    gathered = jnp.take(x, src, axis=0) * edge_weight[:, None]
    return jax.ops.segment_sum(gathered, dst, num_segments=N)


def kernel(x_in, edge_index, edge_weight, W1, b1, W2, b2, gamma, beta, W3, b3, W4, b4):
    src = edge_index[0]
    dst = edge_index[1]
    xin_pad = jnp.pad(x_in, ((0, 0), (0, 1)))
    w1t = jnp.pad(W1.T, ((0, 1), (0, 0)))
    w2t = W2.T
    w2ta = w2t[:H]
    w2tb = w2t[H:]
    b1r = b1.reshape(1, H)
    b2r = b2.reshape(1, H)
    gr = gamma.reshape(1, H)
    br = beta.reshape(1, H)
    w3p = jnp.pad(W3.T, ((0, 0), (0, 7)))
    w4p = jnp.pad(W4.T, ((0, 0), (0, 7)))

    pk, wg = _edge_plumbing(src, dst, edge_weight)

    x, xc = _dense_prologue(xin_pad, w1t, b1r)
    x_init = x
    for _ in range(N_REPEAT):
        xf = xc.reshape(NCH * N, FC)
        neic = _nei_sum_sc(xf, pk, wg)
        nei = neic.transpose(1, 0, 2).reshape(N, H)
        x, xc = _dense_round(x, nei, w2ta, w2tb, b2r, gr, br)
    y = _dense_epilogue(x, x_init, xin_pad, w3p, w4p)
    out = y[:, 0:1] + b3.reshape(1, 1) * x_in[:, 0:1] + b4.reshape(1, 1)
    return out
